# Initial kernel scaffold; baseline (speedup 1.0000x reference)
import jax
import jax.numpy as jnp
from jax.experimental import pallas as pl


def kernel(device_pos, ap_pos, csi, params, node_packets, dd_edge_index, da_src, da_dst, ad_src, ad_dst):
    raise NotImplementedError("write your pallas kernel here")


# speedup vs baseline: 1.9712x; 1.9712x over previous
import functools

import numpy as np
import jax
import jax.numpy as jnp
from jax import lax
from jax.experimental import pallas as pl
from jax.experimental.pallas import tpu as pltpu
from jax.experimental.pallas import tpu_sc as plsc

N_DEV = 10000
N_AP = 8
NP = 10240
APP = 16
N_DD = 640000
N_DA = 80000
DD_P = 655360
DA_P = 81920
NC, NS = 2, 16
STRIPE = NP // NS
B = 512
CHUNK = 128

PAD_SRC_DEV = NP - 1
PAD_DST_DEV = NP - 4
PAD_AP = APP - 1


def _conv_mat_indices(ci_n, co_n):
    rows, cols, widx = [], [], []
    for co in range(co_n):
        for ci in range(ci_n):
            for ky in range(3):
                for kx in range(3):
                    for ho in range(8):
                        hi = ho + ky - 1
                        if not 0 <= hi < 8:
                            continue
                        for wo in range(4):
                            wi = wo + kx - 1
                            if not 0 <= wi < 4:
                                continue
                            rows.append(ci * 32 + hi * 4 + wi)
                            cols.append(co * 32 + ho * 4 + wo)
                            widx.append(((co * ci_n + ci) * 3 + ky) * 3 + kx)
    return (np.asarray(rows, np.int32), np.asarray(cols, np.int32),
            np.asarray(widx, np.int32))


_C1_IDX = _conv_mat_indices(2, 16)
_C2_IDX = _conv_mat_indices(16, 32)

_S_NP = np.zeros((32, 4), np.float32)
for _t in range(4):
    _S_NP[_t * 8:(_t + 1) * 8, _t] = 1.0


def _conv_as_matrix(w, idx, ci_n, co_n):
    r, c, wi = idx
    return jnp.zeros((ci_n * 32, co_n * 32), jnp.float32).at[r, c].set(
        w.reshape(-1)[wi])


def _pad_edges(arr, total, pad_val):
    return jnp.concatenate(
        [arr.astype(jnp.int32),
         jnp.full((total - arr.shape[0],), pad_val, jnp.int32)])


def _sc_mesh():
    return plsc.VectorSubcoreMesh(core_axis_name="c", subcore_axis_name="s",
                                  num_cores=NC, num_subcores=NS)


_SC_PARAMS = pltpu.CompilerParams(use_tc_tiling_on_sc=False)


def _sc_histograms(dd_s, dd_d, da_s, da_d, ad_s, ad_d, zeros8, ones8):
    big = jax.ShapeDtypeStruct((NC, NP, 8), jnp.float32)
    small = jax.ShapeDtypeStruct((NC, APP, 8), jnp.float32)

    @functools.partial(
        pl.kernel,
        out_type=[big, big, big, big, small, small],
        mesh=_sc_mesh(),
        compiler_params=_SC_PARAMS,
        scratch_types=[
            pltpu.VMEM((CHUNK,), jnp.int32),
            pltpu.VMEM((CHUNK, 8), jnp.float32),
            pltpu.VMEM_SHARED((NP, 8), jnp.float32),
            pltpu.VMEM_SHARED((NP, 8), jnp.float32),
            pltpu.VMEM_SHARED((NP, 8), jnp.float32),
            pltpu.VMEM_SHARED((NP, 8), jnp.float32),
            pltpu.VMEM_SHARED((APP, 8), jnp.float32),
            pltpu.VMEM_SHARED((APP, 8), jnp.float32),
        ],
    )
    def body(dds_r, ddd_r, das_r, dad_r, ads_r, add_r, z8_r, o8_r,
             o_dds, o_ddd, o_das, o_add, o_dad, o_ads,
             idx_v, ones_v, a_dds, a_ddd, a_das, a_add, a_dad, a_ads):
        c = lax.axis_index("c")
        s = lax.axis_index("s")
        w = c * NS + s
        rs = s * STRIPE
        for acc in (a_dds, a_ddd, a_das, a_add):
            pltpu.sync_copy(z8_r.at[pl.ds(rs, STRIPE)],
                            acc.at[pl.ds(rs, STRIPE)])

        @pl.when(s == 0)
        def _():
            pltpu.sync_copy(z8_r.at[pl.ds(0, APP)], a_dad)
            pltpu.sync_copy(z8_r.at[pl.ds(0, APP)], a_ads)

        pltpu.sync_copy(o8_r, ones_v)
        plsc.subcore_barrier()

        def histo(arr_r, acc, per_w, nch):
            base0 = w * per_w

            def step(i, carry):
                b = base0 + i * CHUNK
                pltpu.sync_copy(arr_r.at[pl.ds(b, CHUNK)], idx_v)
                pltpu.sync_copy(ones_v, acc.at[idx_v], add=True)
                return carry

            lax.fori_loop(0, nch, step, 0)

        histo(dds_r, a_dds, DD_P // 32, DD_P // 32 // CHUNK)
        histo(ddd_r, a_ddd, DD_P // 32, DD_P // 32 // CHUNK)
        histo(das_r, a_das, DA_P // 32, DA_P // 32 // CHUNK)
        histo(dad_r, a_dad, DA_P // 32, DA_P // 32 // CHUNK)
        histo(ads_r, a_ads, DA_P // 32, DA_P // 32 // CHUNK)
        histo(add_r, a_add, DA_P // 32, DA_P // 32 // CHUNK)
        plsc.subcore_barrier()

        for acc, out in ((a_dds, o_dds), (a_ddd, o_ddd),
                         (a_das, o_das), (a_add, o_add)):
            pltpu.sync_copy(acc.at[pl.ds(rs, STRIPE)],
                            out.at[c, pl.ds(rs, STRIPE)])

        @pl.when(s == 0)
        def _():
            pltpu.sync_copy(a_dad, o_dad.at[c])
            pltpu.sync_copy(a_ads, o_ads.at[c])

    return body(dd_s, dd_d, da_s, da_d, ad_s, ad_d, zeros8, ones8)


def _sc_segsum(feat_dd, feat_da, feat_ad, dd_s, dd_d, da_s, da_d, ad_s, ad_d,
               zeros64, include_da):
    outs = [jax.ShapeDtypeStruct((NC, NP, 64), jnp.float32),
            jax.ShapeDtypeStruct((NC, NP, 64), jnp.float32)]
    if include_da:
        outs.append(jax.ShapeDtypeStruct((NC, APP, 64), jnp.float32))

    @functools.partial(
        pl.kernel,
        out_type=outs,
        mesh=_sc_mesh(),
        compiler_params=_SC_PARAMS,
        scratch_types=[
            pltpu.VMEM((CHUNK,), jnp.int32),
            pltpu.VMEM((CHUNK,), jnp.int32),
            pltpu.VMEM((CHUNK, 64), jnp.float32),
            pltpu.VMEM_SHARED((NP, 64), jnp.float32),
            pltpu.VMEM_SHARED((NP, 64), jnp.float32),
            pltpu.VMEM_SHARED((APP, 64), jnp.float32),
            pltpu.SemaphoreType.DMA,
        ],
    )
    def body(fdd_r, fda_r, fad_r, dds_r, ddd_r, das_r, dad_r, ads_r, add_r,
             z_r, *rest):
        if include_da:
            o_dd, o_ad, o_da = rest[0], rest[1], rest[2]
            rest = rest[3:]
        else:
            o_dd, o_ad = rest[0], rest[1]
            o_da = None
            rest = rest[2:]
        src_v, dst_v, rows_v, a_dd, a_ad, a_da, sem = rest
        c = lax.axis_index("c")
        s = lax.axis_index("s")
        w = c * NS + s
        rs = s * STRIPE
        pltpu.sync_copy(z_r.at[pl.ds(rs, STRIPE)], a_dd.at[pl.ds(rs, STRIPE)])
        pltpu.sync_copy(z_r.at[pl.ds(rs, STRIPE)], a_ad.at[pl.ds(rs, STRIPE)])

        @pl.when(s == 0)
        def _():
            pltpu.sync_copy(z_r.at[pl.ds(0, APP)], a_da)

        plsc.subcore_barrier()

        def relation(src_r, dst_r, table_r, acc, per_w, nch):
            base0 = w * per_w

            def step(i, carry):
                b = base0 + i * CHUNK
                pltpu.sync_copy(src_r.at[pl.ds(b, CHUNK)], src_v)
                pltpu.sync_copy(dst_r.at[pl.ds(b, CHUNK)], dst_v)
                pltpu.async_copy(table_r.at[src_v], rows_v, sem).wait()
                pltpu.sync_copy(rows_v, acc.at[dst_v], add=True)
                return carry

            lax.fori_loop(0, nch, step, 0)

        relation(dds_r, ddd_r, fdd_r, a_dd, DD_P // 32, DD_P // 32 // CHUNK)
        relation(ads_r, add_r, fad_r, a_ad, DA_P // 32, DA_P // 32 // CHUNK)
        if include_da:
            relation(das_r, dad_r, fda_r, a_da, DA_P // 32,
                     DA_P // 32 // CHUNK)
        plsc.subcore_barrier()

        pltpu.sync_copy(a_dd.at[pl.ds(rs, STRIPE)], o_dd.at[c, pl.ds(rs, STRIPE)])
        pltpu.sync_copy(a_ad.at[pl.ds(rs, STRIPE)], o_ad.at[c, pl.ds(rs, STRIPE)])
        if include_da:
            @pl.when(s == 0)
            def _():
                pltpu.sync_copy(a_da, o_da.at[c])

    return body(feat_dd, feat_da, feat_ad, dd_s, dd_d, da_s, da_d, ad_s, ad_d,
                zeros64)


def _dot(a, b):
    return jnp.dot(a, b, preferred_element_type=jnp.float32,
                   precision=lax.Precision.HIGHEST)


def _full(shape):
    return pl.BlockSpec(shape, lambda i: tuple(0 for _ in shape))


def _tc_encoder(x0, pos_in, w1b, b1b, w2b, b2b, wfc, bfc, wdc, bdc,
                wdp, bdp, wdm_p, wdm_c, bdm):
    def body(x0_r, pos_r, w1_r, b1_r, w2_r, b2_r, wf_r, bf_r, wdc_r, bdc_r,
             wdp_r, bdp_r, wmp_r, wmc_r, bdm_r, out_r):
        pos_f = jax.nn.relu(_dot(pos_r[...], wdp_r[...]) + bdp_r[...])
        csum = jnp.zeros((B, 64), jnp.float32)
        for a in range(8):
            xa = x0_r[a]
            h1 = jax.nn.relu(_dot(xa, w1_r[...]) + b1_r[...])
            h2 = jax.nn.relu(_dot(h1, w2_r[...]) + b2_r[...])
            e = jax.nn.relu(_dot(h2, wf_r[...]) + bf_r[...])
            csum = csum + _dot(e, wdc_r[a * 64:(a + 1) * 64, :])
        csi_f = jax.nn.relu(csum + bdc_r[...])
        out_r[...] = jax.nn.relu(
            _dot(pos_f, wmp_r[...]) + _dot(csi_f, wmc_r[...]) + bdm_r[...])

    return pl.pallas_call(
        body,
        grid=(NP // B,),
        in_specs=[
            pl.BlockSpec((8, B, 64), lambda i: (0, i, 0)),
            pl.BlockSpec((B, 8), lambda i: (i, 0)),
            _full((64, 512)), _full((1, 512)),
            _full((512, 1024)), _full((1, 1024)),
            _full((1024, 64)), _full((1, 64)),
            _full((512, 64)), _full((1, 64)),
            _full((8, 16)), _full((1, 16)),
            _full((16, 64)), _full((64, 64)), _full((1, 64)),
        ],
        out_specs=pl.BlockSpec((B, 64), lambda i: (i, 0)),
        out_shape=jax.ShapeDtypeStruct((NP, 64), jnp.float32),
    )(x0, pos_in, w1b, b1b, w2b, b2b, wfc, bfc, wdc, bdc, wdp, bdp,
      wdm_p, wdm_c, bdm)


def _deg_scale(hist):
    deg = hist[0, :, 0:1] + hist[1, :, 0:1]
    return lax.rsqrt(jnp.maximum(deg, 1.0))


def _blk3(minor):
    return pl.BlockSpec((NC, B, minor), lambda i: (0, i, 0))


def _rowblk(minor):
    return pl.BlockSpec((B, minor), lambda i: (i, 0))


def _tc_feat1(h_dev, ap_pos_p, h_dds, h_das, h_ads, w_dd, w_da, w_ap, b_ap,
              w_ad):
    def body(h_r, app_r, hdds_r, hdas_r, hads_r, wdd_r, wda_r, wap_r, bap_r,
             wad_r, o_dd, o_da, o_ad):
        h = h_r[...]
        o_dd[...] = _dot(h * _deg_scale(hdds_r[...]), wdd_r[...])
        o_da[...] = _dot(h * _deg_scale(hdas_r[...]), wda_r[...])

        @pl.when(pl.program_id(0) == 0)
        def _():
            h_ap = jax.nn.relu(_dot(app_r[...], wap_r[...]) + bap_r[...])
            o_ad[...] = _dot(h_ap * _deg_scale(hads_r[...]), wad_r[...])

    return pl.pallas_call(
        body,
        grid=(NP // B,),
        in_specs=[
            _rowblk(64), _full((APP, 8)), _blk3(8), _blk3(8),
            _full((NC, APP, 8)), _full((64, 64)), _full((64, 64)),
            _full((8, 64)), _full((1, 64)), _full((64, 64)),
        ],
        out_specs=[_rowblk(64), _rowblk(64), _full((APP, 64))],
        out_shape=[jax.ShapeDtypeStruct((NP, 64), jnp.float32),
                   jax.ShapeDtypeStruct((NP, 64), jnp.float32),
                   jax.ShapeDtypeStruct((APP, 64), jnp.float32)],
    )(h_dev, ap_pos_p, h_dds, h_das, h_ads, w_dd, w_da, w_ap, b_ap, w_ad)


def _tc_comb1(agg_dd, agg_ad, agg_da, h_ddd, h_add, h_dad, h_dds, h_das,
              h_ads, b_dd, b_ad, b_da, w2dd, w2da, w2ad):
    def body(add_r, aad_r, ada_r, hddd_r, hadd_r, hdad_r, hdds_r, hdas_r,
             hads_r, bdd_r, bad_r, bda_r, wdd_r, wda_r, wad_r,
             o_dd, o_da, o_ad):
        d1 = jax.nn.relu(
            (add_r[0] + add_r[1]) * _deg_scale(hddd_r[...]) + bdd_r[...]
            + (aad_r[0] + aad_r[1]) * _deg_scale(hadd_r[...]) + bad_r[...])
        o_dd[...] = _dot(d1 * _deg_scale(hdds_r[...]), wdd_r[...])
        o_da[...] = _dot(d1 * _deg_scale(hdas_r[...]), wda_r[...])

        @pl.when(pl.program_id(0) == 0)
        def _():
            a1 = jax.nn.relu(
                (ada_r[0] + ada_r[1]) * _deg_scale(hdad_r[...]) + bda_r[...])
            o_ad[...] = _dot(a1 * _deg_scale(hads_r[...]), wad_r[...])

    return pl.pallas_call(
        body,
        grid=(NP // B,),
        in_specs=[
            _blk3(64), _blk3(64), _full((NC, APP, 64)),
            _blk3(8), _blk3(8), _full((NC, APP, 8)),
            _blk3(8), _blk3(8), _full((NC, APP, 8)),
            _full((1, 64)), _full((1, 64)), _full((1, 64)),
            _full((64, 64)), _full((64, 64)), _full((64, 64)),
        ],
        out_specs=[_rowblk(64), _rowblk(64), _full((APP, 64))],
        out_shape=[jax.ShapeDtypeStruct((NP, 64), jnp.float32),
                   jax.ShapeDtypeStruct((NP, 64), jnp.float32),
                   jax.ShapeDtypeStruct((APP, 64), jnp.float32)],
    )(agg_dd, agg_ad, agg_da, h_ddd, h_add, h_dad, h_dds, h_das, h_ads,
      b_dd, b_ad, b_da, w2dd, w2da, w2ad)


BIG_I = 2 ** 30


def _tc_head1(agg_dd, agg_ad, h_ddd, h_add, b_dd, b_ad, w_h1, b_h1, w_h2,
              b_h2):
    def body(add_r, aad_r, hddd_r, hadd_r, bdd_r, bad_r, wh1_r, bh1_r,
             wh2_r, bh2_r, o_logits, o_m, o_s, o_amin):
        i = pl.program_id(0)
        d2 = jax.nn.relu(
            (add_r[0] + add_r[1]) * _deg_scale(hddd_r[...]) + bdd_r[...]
            + (aad_r[0] + aad_r[1]) * _deg_scale(hadd_r[...]) + bad_r[...])
        t = jax.nn.relu(_dot(d2, wh1_r[...]) + bh1_r[...])
        lg = _dot(t, wh2_r[...]) + bh2_r[...]
        o_logits[...] = lg
        rid = lax.broadcasted_iota(jnp.int32, (B, 32), 0) + i * B
        lm = jnp.where(rid < N_DEV, lg, -1e30)
        bm = jnp.max(lm, axis=0, keepdims=True)
        bcand = jnp.min(jnp.where(lm == bm, rid, BIG_I), axis=0,
                        keepdims=True)

        @pl.when(i == 0)
        def _():
            o_m[...] = jnp.full((1, 32), -jnp.inf, jnp.float32)
            o_s[...] = jnp.zeros((1, 32), jnp.float32)
            o_amin[...] = jnp.full((1, 32), BIG_I, jnp.int32)

        pm = o_m[...]
        mn = jnp.maximum(pm, bm)
        o_s[...] = (o_s[...] * jnp.exp(pm - mn)
                    + jnp.sum(jnp.exp(lm - mn), axis=0, keepdims=True))
        prev = o_amin[...]
        o_amin[...] = jnp.where(
            bm > pm, bcand,
            jnp.where(bm == pm, jnp.minimum(prev, bcand), prev))
        o_m[...] = mn

    return pl.pallas_call(
        body,
        grid=(NP // B,),
        in_specs=[
            _blk3(64), _blk3(64), _blk3(8), _blk3(8),
            _full((1, 64)), _full((1, 64)),
            _full((64, 64)), _full((1, 64)), _full((64, 32)), _full((1, 32)),
        ],
        out_specs=[_rowblk(32), _full((1, 32)), _full((1, 32)),
                   _full((1, 32))],
        out_shape=[jax.ShapeDtypeStruct((NP, 32), jnp.float32),
                   jax.ShapeDtypeStruct((1, 32), jnp.float32),
                   jax.ShapeDtypeStruct((1, 32), jnp.float32),
                   jax.ShapeDtypeStruct((1, 32), jnp.int32)],
    )(agg_dd, agg_ad, h_ddd, h_add, b_dd, b_ad, w_h1, b_h1, w_h2, b_h2)


def _tc_head2(logits, m, s, amin, sel):
    def body(lg_r, m_r, s_r, amin_r, sel_r, o_hard, o_soft):
        i = pl.program_id(0)
        lg = lg_r[...]
        rid = lax.broadcasted_iota(jnp.int32, (B, 32), 0) + i * B
        lm = jnp.where(rid < N_DEV, lg, -1e30)
        soft = jnp.exp(lm - m_r[...]) / s_r[...]
        onehot = (rid == amin_r[...]).astype(jnp.float32)
        o_soft[...] = _dot(soft, sel_r[...])
        o_hard[...] = (_dot(onehot, sel_r[...]) > 0).astype(jnp.float32)

    return pl.pallas_call(
        body,
        grid=(NP // B,),
        in_specs=[_rowblk(32), _full((1, 32)), _full((1, 32)),
                  _full((1, 32)), _full((32, 4))],
        out_specs=[_rowblk(4), _rowblk(4)],
        out_shape=[jax.ShapeDtypeStruct((NP, 4), jnp.float32),
                   jax.ShapeDtypeStruct((NP, 4), jnp.float32)],
    )(logits, m, s, amin, sel)


def kernel(device_pos, ap_pos, csi, params, node_packets, dd_edge_index,
           da_src, da_dst, ad_src, ad_dst):
    p = params
    f32 = jnp.float32

    x0 = jnp.transpose(csi, (1, 0, 4, 2, 3)).reshape(N_AP, N_DEV, 64)
    x0 = jnp.pad(x0, ((0, 0), (0, NP - N_DEV), (0, 0)))

    pos_in = jnp.concatenate(
        [device_pos, node_packets[:, None].astype(f32),
         jnp.zeros((N_DEV, 5), f32)], axis=1)
    pos_in = jnp.pad(pos_in, ((0, NP - N_DEV), (0, 0)))

    ap_pos_p = jnp.pad(ap_pos, ((0, APP - N_AP), (0, 6)))

    w1b = _conv_as_matrix(p['csi_conv1_w'], _C1_IDX, 2, 16)
    b1b = jnp.repeat(p['csi_conv1_b'], 32)[None, :]
    w2b = _conv_as_matrix(p['csi_conv2_w'], _C2_IDX, 16, 32)
    b2b = jnp.repeat(p['csi_conv2_b'], 32)[None, :]
    wdp = jnp.pad(p['dev_pos_w'], ((0, 5), (0, 0)))
    wap = jnp.pad(p['ap_enc_w'], ((0, 6), (0, 0)))

    dd_s = _pad_edges(dd_edge_index[0], DD_P, PAD_SRC_DEV)
    dd_d = _pad_edges(dd_edge_index[1], DD_P, PAD_DST_DEV)
    da_s = _pad_edges(da_src, DA_P, PAD_SRC_DEV)
    da_d = _pad_edges(da_dst, DA_P, PAD_AP)
    ad_s = _pad_edges(ad_src, DA_P, PAD_AP)
    ad_d = _pad_edges(ad_dst, DA_P, PAD_DST_DEV)

    zeros8 = jnp.zeros((NP, 8), f32)
    zeros64 = jnp.zeros((NP, 64), f32)
    ones8 = jnp.ones((CHUNK, 8), f32)
    sel = jnp.asarray(_S_NP)

    h_dds, h_ddd, h_das, h_add, h_dad, h_ads = _sc_histograms(
        dd_s, dd_d, da_s, da_d, ad_s, ad_d, zeros8, ones8)

    h_dev = _tc_encoder(
        x0, pos_in, w1b, b1b, w2b, b2b,
        p['csi_fc_w'], p['csi_fc_b'][None, :],
        p['dev_csi_w'], p['dev_csi_b'][None, :],
        wdp, p['dev_pos_b'][None, :],
        p['dev_merge_w'][:16], p['dev_merge_w'][16:],
        p['dev_merge_b'][None, :])

    f_dd1, f_da1, f_ad1 = _tc_feat1(
        h_dev, ap_pos_p, h_dds, h_das, h_ads,
        p['c1_dd_w'], p['c1_da_w'], wap, p['ap_enc_b'][None, :],
        p['c1_ad_w'])
    agg_dd1, agg_ad1, agg_da1 = _sc_segsum(
        f_dd1, f_da1, f_ad1, dd_s, dd_d, da_s, da_d, ad_s, ad_d, zeros64,
        include_da=True)

    f_dd2, f_da2, f_ad2 = _tc_comb1(
        agg_dd1, agg_ad1, agg_da1, h_ddd, h_add, h_dad, h_dds, h_das, h_ads,
        p['c1_dd_b'][None, :], p['c1_ad_b'][None, :], p['c1_da_b'][None, :],
        p['c2_dd_w'], p['c2_da_w'], p['c2_ad_w'])
    agg_dd2, agg_ad2 = _sc_segsum(
        f_dd2, f_da2, f_ad2, dd_s, dd_d, da_s, da_d, ad_s, ad_d, zeros64,
        include_da=False)

    logits_p, m, s, amin = _tc_head1(
        agg_dd2, agg_ad2, h_ddd, h_add,
        p['c2_dd_b'][None, :], p['c2_ad_b'][None, :],
        p['aph1_w'], p['aph1_b'][None, :], p['aph2_w'], p['aph2_b'][None, :])
    hard_p, soft_p = _tc_head2(logits_p, m, s, amin, sel)

    sched_hard = hard_p[:N_DEV]
    ap_logits = logits_p[:N_DEV].reshape(N_DEV, 4, 8)
    sched_soft = soft_p[:N_DEV]
    return sched_hard, ap_logits, sched_soft


# speedup vs baseline: 2.5672x; 1.3023x over previous
import functools

import numpy as np
import jax
import jax.numpy as jnp
from jax import lax
from jax.experimental import pallas as pl
from jax.experimental.pallas import tpu as pltpu
from jax.experimental.pallas import tpu_sc as plsc

N_DEV = 10000
N_AP = 8
NP = 10240
APP = 16
N_DD = 640000
N_DA = 80000
DD_P = 655360
DA_P = 81920
NC, NS = 2, 16
STRIPE = NP // NS
B = 512
CHUNK = 128

PAD_SRC_DEV = NP - 1
PAD_DST_DEV = NP - 4
PAD_AP = APP - 1


def _conv_mat_indices(ci_n, co_n):
    rows, cols, widx = [], [], []
    for co in range(co_n):
        for ci in range(ci_n):
            for ky in range(3):
                for kx in range(3):
                    for ho in range(8):
                        hi = ho + ky - 1
                        if not 0 <= hi < 8:
                            continue
                        for wo in range(4):
                            wi = wo + kx - 1
                            if not 0 <= wi < 4:
                                continue
                            rows.append(ci * 32 + hi * 4 + wi)
                            cols.append(co * 32 + ho * 4 + wo)
                            widx.append(((co * ci_n + ci) * 3 + ky) * 3 + kx)
    return (np.asarray(rows, np.int32), np.asarray(cols, np.int32),
            np.asarray(widx, np.int32))


_C1_IDX = _conv_mat_indices(2, 16)
_C2_IDX = _conv_mat_indices(16, 32)

_S_NP = np.zeros((32, 4), np.float32)
for _t in range(4):
    _S_NP[_t * 8:(_t + 1) * 8, _t] = 1.0


def _conv_as_matrix(w, idx, ci_n, co_n):
    r, c, wi = idx
    return jnp.zeros((ci_n * 32, co_n * 32), jnp.float32).at[r, c].set(
        w.reshape(-1)[wi])


def _pad_edges(arr, total, pad_val):
    return jnp.concatenate(
        [arr.astype(jnp.int32),
         jnp.full((total - arr.shape[0],), pad_val, jnp.int32)])


def _sc_mesh():
    return plsc.VectorSubcoreMesh(core_axis_name="c", subcore_axis_name="s",
                                  num_cores=NC, num_subcores=NS)


_SC_PARAMS = pltpu.CompilerParams(use_tc_tiling_on_sc=False)


def _sc_histograms(dd_s, dd_d, da_s, da_d, ad_s, ad_d, zeros8, ones8):
    big = jax.ShapeDtypeStruct((NC, NP, 8), jnp.float32)
    small = jax.ShapeDtypeStruct((NC, APP, 8), jnp.float32)

    @functools.partial(
        pl.kernel,
        out_type=[big, big, big, big, small, small],
        mesh=_sc_mesh(),
        compiler_params=_SC_PARAMS,
        scratch_types=[
            pltpu.VMEM((CHUNK,), jnp.int32),
            pltpu.VMEM((CHUNK, 8), jnp.float32),
            pltpu.VMEM_SHARED((NP, 8), jnp.float32),
            pltpu.VMEM_SHARED((NP, 8), jnp.float32),
            pltpu.VMEM_SHARED((NP, 8), jnp.float32),
            pltpu.VMEM_SHARED((NP, 8), jnp.float32),
            pltpu.VMEM_SHARED((APP, 8), jnp.float32),
            pltpu.VMEM_SHARED((APP, 8), jnp.float32),
        ],
    )
    def body(dds_r, ddd_r, das_r, dad_r, ads_r, add_r, z8_r, o8_r,
             o_dds, o_ddd, o_das, o_add, o_dad, o_ads,
             idx_v, ones_v, a_dds, a_ddd, a_das, a_add, a_dad, a_ads):
        c = lax.axis_index("c")
        s = lax.axis_index("s")
        w = c * NS + s
        rs = s * STRIPE
        for acc in (a_dds, a_ddd, a_das, a_add):
            pltpu.sync_copy(z8_r.at[pl.ds(rs, STRIPE)],
                            acc.at[pl.ds(rs, STRIPE)])

        @pl.when(s == 0)
        def _():
            pltpu.sync_copy(z8_r.at[pl.ds(0, APP)], a_dad)
            pltpu.sync_copy(z8_r.at[pl.ds(0, APP)], a_ads)

        pltpu.sync_copy(o8_r, ones_v)
        plsc.subcore_barrier()

        def histo(arr_r, acc, per_w, nch):
            base0 = w * per_w

            def step(i, carry):
                b = base0 + i * CHUNK
                pltpu.sync_copy(arr_r.at[pl.ds(b, CHUNK)], idx_v)
                pltpu.sync_copy(ones_v, acc.at[idx_v], add=True)
                return carry

            lax.fori_loop(0, nch, step, 0)

        histo(dds_r, a_dds, DD_P // 32, DD_P // 32 // CHUNK)
        histo(ddd_r, a_ddd, DD_P // 32, DD_P // 32 // CHUNK)
        histo(das_r, a_das, DA_P // 32, DA_P // 32 // CHUNK)
        histo(dad_r, a_dad, DA_P // 32, DA_P // 32 // CHUNK)
        histo(ads_r, a_ads, DA_P // 32, DA_P // 32 // CHUNK)
        histo(add_r, a_add, DA_P // 32, DA_P // 32 // CHUNK)
        plsc.subcore_barrier()

        for acc, out in ((a_dds, o_dds), (a_ddd, o_ddd),
                         (a_das, o_das), (a_add, o_add)):
            pltpu.sync_copy(acc.at[pl.ds(rs, STRIPE)],
                            out.at[c, pl.ds(rs, STRIPE)])

        @pl.when(s == 0)
        def _():
            pltpu.sync_copy(a_dad, o_dad.at[c])
            pltpu.sync_copy(a_ads, o_ads.at[c])

    return body(dd_s, dd_d, da_s, da_d, ad_s, ad_d, zeros8, ones8)


def _sc_segsum(feat_dd, feat_da, feat_ad, dd_s, dd_d, da_s, da_d, ad_s, ad_d,
               zeros64, include_da):
    outs = [jax.ShapeDtypeStruct((NC, NP, 64), jnp.float32),
            jax.ShapeDtypeStruct((NC, NP, 64), jnp.float32)]
    if include_da:
        outs.append(jax.ShapeDtypeStruct((NC, APP, 64), jnp.float32))

    @functools.partial(
        pl.kernel,
        out_type=outs,
        mesh=_sc_mesh(),
        compiler_params=_SC_PARAMS,
        scratch_types=[
            pltpu.VMEM((CHUNK,), jnp.int32),
            pltpu.VMEM((CHUNK,), jnp.int32),
            pltpu.VMEM((CHUNK, 64), jnp.float32),
            pltpu.VMEM_SHARED((NP, 64), jnp.float32),
            pltpu.VMEM_SHARED((NP, 64), jnp.float32),
            pltpu.VMEM_SHARED((APP, 64), jnp.float32),
            pltpu.SemaphoreType.DMA,
        ],
    )
    def body(fdd_r, fda_r, fad_r, dds_r, ddd_r, das_r, dad_r, ads_r, add_r,
             z_r, *rest):
        if include_da:
            o_dd, o_ad, o_da = rest[0], rest[1], rest[2]
            rest = rest[3:]
        else:
            o_dd, o_ad = rest[0], rest[1]
            o_da = None
            rest = rest[2:]
        src_v, dst_v, rows_v, a_dd, a_ad, a_da, sem = rest
        c = lax.axis_index("c")
        s = lax.axis_index("s")
        w = c * NS + s
        rs = s * STRIPE
        pltpu.sync_copy(z_r.at[pl.ds(rs, STRIPE)], a_dd.at[pl.ds(rs, STRIPE)])
        pltpu.sync_copy(z_r.at[pl.ds(rs, STRIPE)], a_ad.at[pl.ds(rs, STRIPE)])

        @pl.when(s == 0)
        def _():
            pltpu.sync_copy(z_r.at[pl.ds(0, APP)], a_da)

        plsc.subcore_barrier()

        def relation(src_r, dst_r, table_r, acc, per_w, nch):
            base0 = w * per_w

            def step(i, carry):
                b = base0 + i * CHUNK
                pltpu.sync_copy(src_r.at[pl.ds(b, CHUNK)], src_v)
                pltpu.sync_copy(dst_r.at[pl.ds(b, CHUNK)], dst_v)
                pltpu.async_copy(table_r.at[src_v], rows_v, sem).wait()
                pltpu.sync_copy(rows_v, acc.at[dst_v], add=True)
                return carry

            lax.fori_loop(0, nch, step, 0)

        relation(dds_r, ddd_r, fdd_r, a_dd, DD_P // 32, DD_P // 32 // CHUNK)
        relation(ads_r, add_r, fad_r, a_ad, DA_P // 32, DA_P // 32 // CHUNK)
        if include_da:
            relation(das_r, dad_r, fda_r, a_da, DA_P // 32,
                     DA_P // 32 // CHUNK)
        plsc.subcore_barrier()

        pltpu.sync_copy(a_dd.at[pl.ds(rs, STRIPE)], o_dd.at[c, pl.ds(rs, STRIPE)])
        pltpu.sync_copy(a_ad.at[pl.ds(rs, STRIPE)], o_ad.at[c, pl.ds(rs, STRIPE)])
        if include_da:
            @pl.when(s == 0)
            def _():
                pltpu.sync_copy(a_da, o_da.at[c])

    return body(feat_dd, feat_da, feat_ad, dd_s, dd_d, da_s, da_d, ad_s, ad_d,
                zeros64)


def _dot(a, b):
    return jnp.dot(a, b, preferred_element_type=jnp.float32)


def _full(shape):
    return pl.BlockSpec(shape, lambda i: tuple(0 for _ in shape))


def _tc_encoder(x0, pos_in, w1b, b1b, w2b, b2b, wfc, bfc, wdc, bdc,
                wdp, bdp, wdm_p, wdm_c, bdm):
    def body(x0_r, pos_r, w1_r, b1_r, w2_r, b2_r, wf_r, bf_r, wdc_r, bdc_r,
             wdp_r, bdp_r, wmp_r, wmc_r, bdm_r, out_r):
        pos_f = jax.nn.relu(_dot(pos_r[...], wdp_r[...]) + bdp_r[...])
        csum = jnp.zeros((B, 64), jnp.float32)
        for a in range(8):
            xa = x0_r[a]
            h1 = jax.nn.relu(_dot(xa, w1_r[...]) + b1_r[...])
            h2 = jax.nn.relu(_dot(h1, w2_r[...]) + b2_r[...])
            e = jax.nn.relu(_dot(h2, wf_r[...]) + bf_r[...])
            csum = csum + _dot(e, wdc_r[a * 64:(a + 1) * 64, :])
        csi_f = jax.nn.relu(csum + bdc_r[...])
        out_r[...] = jax.nn.relu(
            _dot(pos_f, wmp_r[...]) + _dot(csi_f, wmc_r[...]) + bdm_r[...])

    return pl.pallas_call(
        body,
        grid=(NP // B,),
        in_specs=[
            pl.BlockSpec((8, B, 64), lambda i: (0, i, 0)),
            pl.BlockSpec((B, 8), lambda i: (i, 0)),
            _full((64, 512)), _full((1, 512)),
            _full((512, 1024)), _full((1, 1024)),
            _full((1024, 64)), _full((1, 64)),
            _full((512, 64)), _full((1, 64)),
            _full((8, 16)), _full((1, 16)),
            _full((16, 64)), _full((64, 64)), _full((1, 64)),
        ],
        out_specs=pl.BlockSpec((B, 64), lambda i: (i, 0)),
        out_shape=jax.ShapeDtypeStruct((NP, 64), jnp.float32),
    )(x0, pos_in, w1b, b1b, w2b, b2b, wfc, bfc, wdc, bdc, wdp, bdp,
      wdm_p, wdm_c, bdm)


def _deg_scale(hist):
    deg = hist[0, :, 0:1] + hist[1, :, 0:1]
    return lax.rsqrt(jnp.maximum(deg, 1.0))


def _blk3(minor):
    return pl.BlockSpec((NC, B, minor), lambda i: (0, i, 0))


def _rowblk(minor):
    return pl.BlockSpec((B, minor), lambda i: (i, 0))


def _tc_feat1(h_dev, ap_pos_p, h_dds, h_das, h_ads, w_ap, b_ap):
    def body(h_r, app_r, hdds_r, hdas_r, hads_r, wap_r, bap_r,
             o_dd, o_da, o_ad):
        h = h_r[...]
        o_dd[...] = h * _deg_scale(hdds_r[...])
        o_da[...] = h * _deg_scale(hdas_r[...])

        @pl.when(pl.program_id(0) == 0)
        def _():
            h_ap = jax.nn.relu(_dot(app_r[...], wap_r[...]) + bap_r[...])
            o_ad[...] = h_ap * _deg_scale(hads_r[...])

    return pl.pallas_call(
        body,
        grid=(NP // B,),
        in_specs=[
            _rowblk(64), _full((APP, 8)), _blk3(8), _blk3(8),
            _full((NC, APP, 8)), _full((8, 64)), _full((1, 64)),
        ],
        out_specs=[_rowblk(64), _rowblk(64), _full((APP, 64))],
        out_shape=[jax.ShapeDtypeStruct((NP, 64), jnp.float32),
                   jax.ShapeDtypeStruct((NP, 64), jnp.float32),
                   jax.ShapeDtypeStruct((APP, 64), jnp.float32)],
    )(h_dev, ap_pos_p, h_dds, h_das, h_ads, w_ap, b_ap)


def _tc_comb1(agg_dd, agg_ad, agg_da, h_ddd, h_add, h_dad, h_dds, h_das,
              h_ads, w1dd, w1ad, w1da, b_dd, b_ad, b_da):
    def body(add_r, aad_r, ada_r, hddd_r, hadd_r, hdad_r, hdds_r, hdas_r,
             hads_r, wdd_r, wad_r, wda_r, bdd_r, bad_r, bda_r,
             o_dd, o_da, o_ad):
        d1 = jax.nn.relu(
            _dot((add_r[0] + add_r[1]) * _deg_scale(hddd_r[...]), wdd_r[...])
            + bdd_r[...]
            + _dot((aad_r[0] + aad_r[1]) * _deg_scale(hadd_r[...]), wad_r[...])
            + bad_r[...])
        o_dd[...] = d1 * _deg_scale(hdds_r[...])
        o_da[...] = d1 * _deg_scale(hdas_r[...])

        @pl.when(pl.program_id(0) == 0)
        def _():
            a1 = jax.nn.relu(
                _dot((ada_r[0] + ada_r[1]) * _deg_scale(hdad_r[...]),
                     wda_r[...]) + bda_r[...])
            o_ad[...] = a1 * _deg_scale(hads_r[...])

    return pl.pallas_call(
        body,
        grid=(NP // B,),
        in_specs=[
            _blk3(64), _blk3(64), _full((NC, APP, 64)),
            _blk3(8), _blk3(8), _full((NC, APP, 8)),
            _blk3(8), _blk3(8), _full((NC, APP, 8)),
            _full((64, 64)), _full((64, 64)), _full((64, 64)),
            _full((1, 64)), _full((1, 64)), _full((1, 64)),
        ],
        out_specs=[_rowblk(64), _rowblk(64), _full((APP, 64))],
        out_shape=[jax.ShapeDtypeStruct((NP, 64), jnp.float32),
                   jax.ShapeDtypeStruct((NP, 64), jnp.float32),
                   jax.ShapeDtypeStruct((APP, 64), jnp.float32)],
    )(agg_dd, agg_ad, agg_da, h_ddd, h_add, h_dad, h_dds, h_das, h_ads,
      w1dd, w1ad, w1da, b_dd, b_ad, b_da)


BIG_I = 2 ** 30


def _tc_head1(agg_dd, agg_ad, h_ddd, h_add, w2dd, w2ad, b_dd, b_ad, w_h1,
              b_h1, w_h2, b_h2):
    def body(add_r, aad_r, hddd_r, hadd_r, wdd_r, wad_r, bdd_r, bad_r,
             wh1_r, bh1_r, wh2_r, bh2_r, o_logits, o_m, o_s, o_amin):
        i = pl.program_id(0)
        d2 = jax.nn.relu(
            _dot((add_r[0] + add_r[1]) * _deg_scale(hddd_r[...]), wdd_r[...])
            + bdd_r[...]
            + _dot((aad_r[0] + aad_r[1]) * _deg_scale(hadd_r[...]), wad_r[...])
            + bad_r[...])
        t = jax.nn.relu(_dot(d2, wh1_r[...]) + bh1_r[...])
        lg = _dot(t, wh2_r[...]) + bh2_r[...]
        o_logits[...] = lg
        rid = lax.broadcasted_iota(jnp.int32, (B, 32), 0) + i * B
        lm = jnp.where(rid < N_DEV, lg, -1e30)
        bm = jnp.max(lm, axis=0, keepdims=True)
        bcand = jnp.min(jnp.where(lm == bm, rid, BIG_I), axis=0,
                        keepdims=True)

        @pl.when(i == 0)
        def _():
            o_m[...] = jnp.full((1, 32), -jnp.inf, jnp.float32)
            o_s[...] = jnp.zeros((1, 32), jnp.float32)
            o_amin[...] = jnp.full((1, 32), BIG_I, jnp.int32)

        pm = o_m[...]
        mn = jnp.maximum(pm, bm)
        o_s[...] = (o_s[...] * jnp.exp(pm - mn)
                    + jnp.sum(jnp.exp(lm - mn), axis=0, keepdims=True))
        prev = o_amin[...]
        o_amin[...] = jnp.where(
            bm > pm, bcand,
            jnp.where(bm == pm, jnp.minimum(prev, bcand), prev))
        o_m[...] = mn

    return pl.pallas_call(
        body,
        grid=(NP // B,),
        in_specs=[
            _blk3(64), _blk3(64), _blk3(8), _blk3(8),
            _full((64, 64)), _full((64, 64)),
            _full((1, 64)), _full((1, 64)),
            _full((64, 64)), _full((1, 64)), _full((64, 32)), _full((1, 32)),
        ],
        out_specs=[_rowblk(32), _full((1, 32)), _full((1, 32)),
                   _full((1, 32))],
        out_shape=[jax.ShapeDtypeStruct((NP, 32), jnp.float32),
                   jax.ShapeDtypeStruct((1, 32), jnp.float32),
                   jax.ShapeDtypeStruct((1, 32), jnp.float32),
                   jax.ShapeDtypeStruct((1, 32), jnp.int32)],
    )(agg_dd, agg_ad, h_ddd, h_add, w2dd, w2ad, b_dd, b_ad, w_h1, b_h1,
      w_h2, b_h2)


def _tc_head2(logits, m, s, amin, sel):
    def body(lg_r, m_r, s_r, amin_r, sel_r, o_hard, o_soft):
        i = pl.program_id(0)
        lg = lg_r[...]
        rid = lax.broadcasted_iota(jnp.int32, (B, 32), 0) + i * B
        lm = jnp.where(rid < N_DEV, lg, -1e30)
        soft = jnp.exp(lm - m_r[...]) / s_r[...]
        onehot = (rid == amin_r[...]).astype(jnp.float32)
        o_soft[...] = _dot(soft, sel_r[...])
        o_hard[...] = (_dot(onehot, sel_r[...]) > 0).astype(jnp.float32)

    return pl.pallas_call(
        body,
        grid=(NP // B,),
        in_specs=[_rowblk(32), _full((1, 32)), _full((1, 32)),
                  _full((1, 32)), _full((32, 4))],
        out_specs=[_rowblk(4), _rowblk(4)],
        out_shape=[jax.ShapeDtypeStruct((NP, 4), jnp.float32),
                   jax.ShapeDtypeStruct((NP, 4), jnp.float32)],
    )(logits, m, s, amin, sel)


def kernel(device_pos, ap_pos, csi, params, node_packets, dd_edge_index,
           da_src, da_dst, ad_src, ad_dst):
    p = params
    f32 = jnp.float32

    x0 = jnp.transpose(csi, (1, 0, 4, 2, 3)).reshape(N_AP, N_DEV, 64)
    x0 = jnp.pad(x0, ((0, 0), (0, NP - N_DEV), (0, 0)))

    pos_in = jnp.concatenate(
        [device_pos, node_packets[:, None].astype(f32),
         jnp.zeros((N_DEV, 5), f32)], axis=1)
    pos_in = jnp.pad(pos_in, ((0, NP - N_DEV), (0, 0)))

    ap_pos_p = jnp.pad(ap_pos, ((0, APP - N_AP), (0, 6)))

    w1b = _conv_as_matrix(p['csi_conv1_w'], _C1_IDX, 2, 16)
    b1b = jnp.repeat(p['csi_conv1_b'], 32)[None, :]
    w2b = _conv_as_matrix(p['csi_conv2_w'], _C2_IDX, 16, 32)
    b2b = jnp.repeat(p['csi_conv2_b'], 32)[None, :]
    wdp = jnp.pad(p['dev_pos_w'], ((0, 5), (0, 0)))
    wap = jnp.pad(p['ap_enc_w'], ((0, 6), (0, 0)))

    dd_s = _pad_edges(dd_edge_index[0], DD_P, PAD_SRC_DEV)
    dd_d = _pad_edges(dd_edge_index[1], DD_P, PAD_DST_DEV)
    da_s = _pad_edges(da_src, DA_P, PAD_SRC_DEV)
    da_d = _pad_edges(da_dst, DA_P, PAD_AP)
    ad_s = _pad_edges(ad_src, DA_P, PAD_AP)
    ad_d = _pad_edges(ad_dst, DA_P, PAD_DST_DEV)

    zeros8 = jnp.zeros((NP, 8), f32)
    zeros64 = jnp.zeros((NP, 64), f32)
    ones8 = jnp.ones((CHUNK, 8), f32)
    sel = jnp.asarray(_S_NP)

    h_dds, h_ddd, h_das, h_add, h_dad, h_ads = _sc_histograms(
        dd_s, dd_d, da_s, da_d, ad_s, ad_d, zeros8, ones8)

    h_dev = _tc_encoder(
        x0, pos_in, w1b, b1b, w2b, b2b,
        p['csi_fc_w'], p['csi_fc_b'][None, :],
        p['dev_csi_w'], p['dev_csi_b'][None, :],
        wdp, p['dev_pos_b'][None, :],
        p['dev_merge_w'][:16], p['dev_merge_w'][16:],
        p['dev_merge_b'][None, :])

    f_dd1, f_da1, f_ad1 = _tc_feat1(
        h_dev, ap_pos_p, h_dds, h_das, h_ads, wap, p['ap_enc_b'][None, :])
    agg_dd1, agg_ad1, agg_da1 = _sc_segsum(
        f_dd1, f_da1, f_ad1, dd_s, dd_d, da_s, da_d, ad_s, ad_d, zeros64,
        include_da=True)

    f_dd2, f_da2, f_ad2 = _tc_comb1(
        agg_dd1, agg_ad1, agg_da1, h_ddd, h_add, h_dad, h_dds, h_das, h_ads,
        p['c1_dd_w'], p['c1_ad_w'], p['c1_da_w'],
        p['c1_dd_b'][None, :], p['c1_ad_b'][None, :], p['c1_da_b'][None, :])
    agg_dd2, agg_ad2 = _sc_segsum(
        f_dd2, f_da2, f_ad2, dd_s, dd_d, da_s, da_d, ad_s, ad_d, zeros64,
        include_da=False)

    logits_p, m, s, amin = _tc_head1(
        agg_dd2, agg_ad2, h_ddd, h_add,
        p['c2_dd_w'], p['c2_ad_w'],
        p['c2_dd_b'][None, :], p['c2_ad_b'][None, :],
        p['aph1_w'], p['aph1_b'][None, :], p['aph2_w'], p['aph2_b'][None, :])
    hard_p, soft_p = _tc_head2(logits_p, m, s, amin, sel)

    sched_hard = hard_p[:N_DEV]
    ap_logits = logits_p[:N_DEV].reshape(N_DEV, 4, 8)
    sched_soft = soft_p[:N_DEV]
    return sched_hard, ap_logits, sched_soft


# speedup vs baseline: 2.7829x; 1.0841x over previous
import functools

import numpy as np
import jax
import jax.numpy as jnp
from jax import lax
from jax.experimental import pallas as pl
from jax.experimental.pallas import tpu as pltpu
from jax.experimental.pallas import tpu_sc as plsc

N_DEV = 10000
N_AP = 8
NP = 10240
APP = 16
N_DD = 640000
N_DA = 80000
DD_P = 655360
DA_P = 81920
NC, NS = 2, 16
STRIPE = NP // NS
B = 512
CHUNK = 128

PAD_SRC_DEV = NP - 1
PAD_DST_DEV = NP - 4
PAD_AP = APP - 1


def _conv_mat_indices(ci_n, co_n):
    rows, cols, widx = [], [], []
    for co in range(co_n):
        for ci in range(ci_n):
            for ky in range(3):
                for kx in range(3):
                    for ho in range(8):
                        hi = ho + ky - 1
                        if not 0 <= hi < 8:
                            continue
                        for wo in range(4):
                            wi = wo + kx - 1
                            if not 0 <= wi < 4:
                                continue
                            rows.append(ci * 32 + hi * 4 + wi)
                            cols.append(co * 32 + ho * 4 + wo)
                            widx.append(((co * ci_n + ci) * 3 + ky) * 3 + kx)
    return (np.asarray(rows, np.int32), np.asarray(cols, np.int32),
            np.asarray(widx, np.int32))


_C1_IDX = _conv_mat_indices(2, 16)
_C2_IDX = _conv_mat_indices(16, 32)

_S_NP = np.zeros((32, 4), np.float32)
for _t in range(4):
    _S_NP[_t * 8:(_t + 1) * 8, _t] = 1.0


def _conv_as_matrix(w, idx, ci_n, co_n):
    r, c, wi = idx
    return jnp.zeros((ci_n * 32, co_n * 32), jnp.float32).at[r, c].set(
        w.reshape(-1)[wi])


def _pad_edges(arr, total, pad_val):
    return jnp.concatenate(
        [arr.astype(jnp.int32),
         jnp.full((total - arr.shape[0],), pad_val, jnp.int32)])


def _sc_mesh():
    return plsc.VectorSubcoreMesh(core_axis_name="c", subcore_axis_name="s",
                                  num_cores=NC, num_subcores=NS)


_SC_PARAMS = pltpu.CompilerParams(use_tc_tiling_on_sc=False)


def _sc_histograms(dd_s, dd_d, da_s, da_d, ad_s, ad_d, zeros8, ones8):
    big = jax.ShapeDtypeStruct((NC, NP, 8), jnp.float32)
    small = jax.ShapeDtypeStruct((NC, APP, 8), jnp.float32)

    @functools.partial(
        pl.kernel,
        out_type=[big, big, big, big, small, small],
        mesh=_sc_mesh(),
        compiler_params=_SC_PARAMS,
        scratch_types=[
            pltpu.VMEM((CHUNK,), jnp.int32),
            pltpu.VMEM((CHUNK, 8), jnp.float32),
            pltpu.VMEM_SHARED((NP, 8), jnp.float32),
            pltpu.VMEM_SHARED((NP, 8), jnp.float32),
            pltpu.VMEM_SHARED((NP, 8), jnp.float32),
            pltpu.VMEM_SHARED((NP, 8), jnp.float32),
            pltpu.VMEM_SHARED((APP, 8), jnp.float32),
            pltpu.VMEM_SHARED((APP, 8), jnp.float32),
        ],
    )
    def body(dds_r, ddd_r, das_r, dad_r, ads_r, add_r, z8_r, o8_r,
             o_dds, o_ddd, o_das, o_add, o_dad, o_ads,
             idx_v, ones_v, a_dds, a_ddd, a_das, a_add, a_dad, a_ads):
        c = lax.axis_index("c")
        s = lax.axis_index("s")
        w = c * NS + s
        rs = s * STRIPE
        for acc in (a_dds, a_ddd, a_das, a_add):
            pltpu.sync_copy(z8_r.at[pl.ds(rs, STRIPE)],
                            acc.at[pl.ds(rs, STRIPE)])

        @pl.when(s == 0)
        def _():
            pltpu.sync_copy(z8_r.at[pl.ds(0, APP)], a_dad)
            pltpu.sync_copy(z8_r.at[pl.ds(0, APP)], a_ads)

        pltpu.sync_copy(o8_r, ones_v)
        plsc.subcore_barrier()

        def histo(arr_r, acc, per_w, nch):
            base0 = w * per_w

            def step(i, carry):
                b = base0 + i * CHUNK
                pltpu.sync_copy(arr_r.at[pl.ds(b, CHUNK)], idx_v)
                pltpu.sync_copy(ones_v, acc.at[idx_v], add=True)
                return carry

            lax.fori_loop(0, nch, step, 0)

        histo(dds_r, a_dds, DD_P // 32, DD_P // 32 // CHUNK)
        histo(ddd_r, a_ddd, DD_P // 32, DD_P // 32 // CHUNK)
        histo(das_r, a_das, DA_P // 32, DA_P // 32 // CHUNK)
        histo(dad_r, a_dad, DA_P // 32, DA_P // 32 // CHUNK)
        histo(ads_r, a_ads, DA_P // 32, DA_P // 32 // CHUNK)
        histo(add_r, a_add, DA_P // 32, DA_P // 32 // CHUNK)
        plsc.subcore_barrier()

        for acc, out in ((a_dds, o_dds), (a_ddd, o_ddd),
                         (a_das, o_das), (a_add, o_add)):
            pltpu.sync_copy(acc.at[pl.ds(rs, STRIPE)],
                            out.at[c, pl.ds(rs, STRIPE)])

        @pl.when(s == 0)
        def _():
            pltpu.sync_copy(a_dad, o_dad.at[c])
            pltpu.sync_copy(a_ads, o_ads.at[c])

    return body(dd_s, dd_d, da_s, da_d, ad_s, ad_d, zeros8, ones8)


def _sc_segsum(feat_dd, feat_da, feat_ad, dd_s, dd_d, da_s, da_d, ad_s, ad_d,
               zeros64, include_da):
    outs = [jax.ShapeDtypeStruct((NC, NP, 64), jnp.float32),
            jax.ShapeDtypeStruct((NC, NP, 64), jnp.float32)]
    if include_da:
        outs.append(jax.ShapeDtypeStruct((NC, APP, 64), jnp.float32))

    @functools.partial(
        pl.kernel,
        out_type=outs,
        mesh=_sc_mesh(),
        compiler_params=_SC_PARAMS,
        scratch_types=[
            pltpu.VMEM((CHUNK,), jnp.int32),
            pltpu.VMEM((CHUNK,), jnp.int32),
            pltpu.VMEM((CHUNK,), jnp.int32),
            pltpu.VMEM((CHUNK,), jnp.int32),
            pltpu.VMEM((CHUNK, 64), jnp.float32),
            pltpu.VMEM((CHUNK, 64), jnp.float32),
            pltpu.VMEM_SHARED((NP, 64), jnp.float32),
            pltpu.VMEM_SHARED((NP, 64), jnp.float32),
            pltpu.VMEM_SHARED((APP, 64), jnp.float32),
            pltpu.SemaphoreType.DMA,
            pltpu.SemaphoreType.DMA,
        ],
    )
    def body(fdd_r, fda_r, fad_r, dds_r, ddd_r, das_r, dad_r, ads_r, add_r,
             z_r, *rest):
        if include_da:
            o_dd, o_ad, o_da = rest[0], rest[1], rest[2]
            rest = rest[3:]
        else:
            o_dd, o_ad = rest[0], rest[1]
            o_da = None
            rest = rest[2:]
        (src0, dst0, src1, dst1, rows0, rows1, a_dd, a_ad, a_da,
         sem0, sem1) = rest
        c = lax.axis_index("c")
        s = lax.axis_index("s")
        w = c * NS + s
        rs = s * STRIPE
        pltpu.sync_copy(z_r.at[pl.ds(rs, STRIPE)], a_dd.at[pl.ds(rs, STRIPE)])
        pltpu.sync_copy(z_r.at[pl.ds(rs, STRIPE)], a_ad.at[pl.ds(rs, STRIPE)])

        @pl.when(s == 0)
        def _():
            pltpu.sync_copy(z_r.at[pl.ds(0, APP)], a_da)

        plsc.subcore_barrier()

        def relation(src_r, dst_r, table_r, acc, per_w, nch):
            base0 = w * per_w

            def step(i2, carry):
                e = base0 + (2 * i2) * CHUNK
                pltpu.sync_copy(src_r.at[pl.ds(e, CHUNK)], src0)
                cp0 = pltpu.async_copy(table_r.at[src0], rows0, sem0)
                pltpu.sync_copy(src_r.at[pl.ds(e + CHUNK, CHUNK)], src1)
                cp1 = pltpu.async_copy(table_r.at[src1], rows1, sem1)
                pltpu.sync_copy(dst_r.at[pl.ds(e, CHUNK)], dst0)
                cp0.wait()
                pltpu.sync_copy(rows0, acc.at[dst0], add=True)
                pltpu.sync_copy(dst_r.at[pl.ds(e + CHUNK, CHUNK)], dst1)
                cp1.wait()
                pltpu.sync_copy(rows1, acc.at[dst1], add=True)
                return carry

            lax.fori_loop(0, nch // 2, step, 0)

        relation(dds_r, ddd_r, fdd_r, a_dd, DD_P // 32, DD_P // 32 // CHUNK)
        relation(ads_r, add_r, fad_r, a_ad, DA_P // 32, DA_P // 32 // CHUNK)
        if include_da:
            relation(das_r, dad_r, fda_r, a_da, DA_P // 32,
                     DA_P // 32 // CHUNK)
        plsc.subcore_barrier()

        pltpu.sync_copy(a_dd.at[pl.ds(rs, STRIPE)], o_dd.at[c, pl.ds(rs, STRIPE)])
        pltpu.sync_copy(a_ad.at[pl.ds(rs, STRIPE)], o_ad.at[c, pl.ds(rs, STRIPE)])
        if include_da:
            @pl.when(s == 0)
            def _():
                pltpu.sync_copy(a_da, o_da.at[c])

    return body(feat_dd, feat_da, feat_ad, dd_s, dd_d, da_s, da_d, ad_s, ad_d,
                zeros64)


def _dot(a, b):
    return jnp.dot(a, b, preferred_element_type=jnp.float32)


def _full(shape):
    return pl.BlockSpec(shape, lambda i: tuple(0 for _ in shape))


def _tc_encoder(x0, pos_in, w1b, b1b, w2b, b2b, wfc, bfc, wdc, bdc,
                wdp, bdp, wdm_p, wdm_c, bdm):
    def body(x0_r, pos_r, w1_r, b1_r, w2_r, b2_r, wf_r, bf_r, wdc_r, bdc_r,
             wdp_r, bdp_r, wmp_r, wmc_r, bdm_r, out_r):
        pos_f = jax.nn.relu(_dot(pos_r[...], wdp_r[...]) + bdp_r[...])
        csum = jnp.zeros((B, 64), jnp.float32)
        for a in range(8):
            xa = x0_r[a]
            h1 = jax.nn.relu(_dot(xa, w1_r[...]) + b1_r[...])
            h2 = jax.nn.relu(_dot(h1, w2_r[...]) + b2_r[...])
            e = jax.nn.relu(_dot(h2, wf_r[...]) + bf_r[...])
            csum = csum + _dot(e, wdc_r[a * 64:(a + 1) * 64, :])
        csi_f = jax.nn.relu(csum + bdc_r[...])
        out_r[...] = jax.nn.relu(
            _dot(pos_f, wmp_r[...]) + _dot(csi_f, wmc_r[...]) + bdm_r[...])

    return pl.pallas_call(
        body,
        grid=(NP // B,),
        in_specs=[
            pl.BlockSpec((8, B, 64), lambda i: (0, i, 0)),
            pl.BlockSpec((B, 8), lambda i: (i, 0)),
            _full((64, 512)), _full((1, 512)),
            _full((512, 1024)), _full((1, 1024)),
            _full((1024, 64)), _full((1, 64)),
            _full((512, 64)), _full((1, 64)),
            _full((8, 16)), _full((1, 16)),
            _full((16, 64)), _full((64, 64)), _full((1, 64)),
        ],
        out_specs=pl.BlockSpec((B, 64), lambda i: (i, 0)),
        out_shape=jax.ShapeDtypeStruct((NP, 64), jnp.float32),
    )(x0, pos_in, w1b, b1b, w2b, b2b, wfc, bfc, wdc, bdc, wdp, bdp,
      wdm_p, wdm_c, bdm)


def _deg_scale(hist):
    deg = hist[0, :, 0:1] + hist[1, :, 0:1]
    return lax.rsqrt(jnp.maximum(deg, 1.0))


def _blk3(minor):
    return pl.BlockSpec((NC, B, minor), lambda i: (0, i, 0))


def _rowblk(minor):
    return pl.BlockSpec((B, minor), lambda i: (i, 0))


def _tc_feat1(h_dev, ap_pos_p, h_dds, h_das, h_ads, w_ap, b_ap):
    def body(h_r, app_r, hdds_r, hdas_r, hads_r, wap_r, bap_r,
             o_dd, o_da, o_ad):
        h = h_r[...]
        o_dd[...] = h * _deg_scale(hdds_r[...])
        o_da[...] = h * _deg_scale(hdas_r[...])

        @pl.when(pl.program_id(0) == 0)
        def _():
            h_ap = jax.nn.relu(_dot(app_r[...], wap_r[...]) + bap_r[...])
            o_ad[...] = h_ap * _deg_scale(hads_r[...])

    return pl.pallas_call(
        body,
        grid=(NP // B,),
        in_specs=[
            _rowblk(64), _full((APP, 8)), _blk3(8), _blk3(8),
            _full((NC, APP, 8)), _full((8, 64)), _full((1, 64)),
        ],
        out_specs=[_rowblk(64), _rowblk(64), _full((APP, 64))],
        out_shape=[jax.ShapeDtypeStruct((NP, 64), jnp.float32),
                   jax.ShapeDtypeStruct((NP, 64), jnp.float32),
                   jax.ShapeDtypeStruct((APP, 64), jnp.float32)],
    )(h_dev, ap_pos_p, h_dds, h_das, h_ads, w_ap, b_ap)


def _tc_comb1(agg_dd, agg_ad, agg_da, h_ddd, h_add, h_dad, h_dds, h_das,
              h_ads, w1dd, w1ad, w1da, b_dd, b_ad, b_da):
    def body(add_r, aad_r, ada_r, hddd_r, hadd_r, hdad_r, hdds_r, hdas_r,
             hads_r, wdd_r, wad_r, wda_r, bdd_r, bad_r, bda_r,
             o_dd, o_da, o_ad):
        d1 = jax.nn.relu(
            _dot((add_r[0] + add_r[1]) * _deg_scale(hddd_r[...]), wdd_r[...])
            + bdd_r[...]
            + _dot((aad_r[0] + aad_r[1]) * _deg_scale(hadd_r[...]), wad_r[...])
            + bad_r[...])
        o_dd[...] = d1 * _deg_scale(hdds_r[...])
        o_da[...] = d1 * _deg_scale(hdas_r[...])

        @pl.when(pl.program_id(0) == 0)
        def _():
            a1 = jax.nn.relu(
                _dot((ada_r[0] + ada_r[1]) * _deg_scale(hdad_r[...]),
                     wda_r[...]) + bda_r[...])
            o_ad[...] = a1 * _deg_scale(hads_r[...])

    return pl.pallas_call(
        body,
        grid=(NP // B,),
        in_specs=[
            _blk3(64), _blk3(64), _full((NC, APP, 64)),
            _blk3(8), _blk3(8), _full((NC, APP, 8)),
            _blk3(8), _blk3(8), _full((NC, APP, 8)),
            _full((64, 64)), _full((64, 64)), _full((64, 64)),
            _full((1, 64)), _full((1, 64)), _full((1, 64)),
        ],
        out_specs=[_rowblk(64), _rowblk(64), _full((APP, 64))],
        out_shape=[jax.ShapeDtypeStruct((NP, 64), jnp.float32),
                   jax.ShapeDtypeStruct((NP, 64), jnp.float32),
                   jax.ShapeDtypeStruct((APP, 64), jnp.float32)],
    )(agg_dd, agg_ad, agg_da, h_ddd, h_add, h_dad, h_dds, h_das, h_ads,
      w1dd, w1ad, w1da, b_dd, b_ad, b_da)


BIG_I = 2 ** 30


def _tc_head1(agg_dd, agg_ad, h_ddd, h_add, w2dd, w2ad, b_dd, b_ad, w_h1,
              b_h1, w_h2, b_h2):
    def body(add_r, aad_r, hddd_r, hadd_r, wdd_r, wad_r, bdd_r, bad_r,
             wh1_r, bh1_r, wh2_r, bh2_r, o_logits, o_m, o_s, o_amin):
        i = pl.program_id(0)
        d2 = jax.nn.relu(
            _dot((add_r[0] + add_r[1]) * _deg_scale(hddd_r[...]), wdd_r[...])
            + bdd_r[...]
            + _dot((aad_r[0] + aad_r[1]) * _deg_scale(hadd_r[...]), wad_r[...])
            + bad_r[...])
        t = jax.nn.relu(_dot(d2, wh1_r[...]) + bh1_r[...])
        lg = _dot(t, wh2_r[...]) + bh2_r[...]
        o_logits[...] = lg
        rid = lax.broadcasted_iota(jnp.int32, (B, 32), 0) + i * B
        lm = jnp.where(rid < N_DEV, lg, -1e30)
        bm = jnp.max(lm, axis=0, keepdims=True)
        bcand = jnp.min(jnp.where(lm == bm, rid, BIG_I), axis=0,
                        keepdims=True)

        @pl.when(i == 0)
        def _():
            o_m[...] = jnp.full((1, 32), -jnp.inf, jnp.float32)
            o_s[...] = jnp.zeros((1, 32), jnp.float32)
            o_amin[...] = jnp.full((1, 32), BIG_I, jnp.int32)

        pm = o_m[...]
        mn = jnp.maximum(pm, bm)
        o_s[...] = (o_s[...] * jnp.exp(pm - mn)
                    + jnp.sum(jnp.exp(lm - mn), axis=0, keepdims=True))
        prev = o_amin[...]
        o_amin[...] = jnp.where(
            bm > pm, bcand,
            jnp.where(bm == pm, jnp.minimum(prev, bcand), prev))
        o_m[...] = mn

    return pl.pallas_call(
        body,
        grid=(NP // B,),
        in_specs=[
            _blk3(64), _blk3(64), _blk3(8), _blk3(8),
            _full((64, 64)), _full((64, 64)),
            _full((1, 64)), _full((1, 64)),
            _full((64, 64)), _full((1, 64)), _full((64, 32)), _full((1, 32)),
        ],
        out_specs=[_rowblk(32), _full((1, 32)), _full((1, 32)),
                   _full((1, 32))],
        out_shape=[jax.ShapeDtypeStruct((NP, 32), jnp.float32),
                   jax.ShapeDtypeStruct((1, 32), jnp.float32),
                   jax.ShapeDtypeStruct((1, 32), jnp.float32),
                   jax.ShapeDtypeStruct((1, 32), jnp.int32)],
    )(agg_dd, agg_ad, h_ddd, h_add, w2dd, w2ad, b_dd, b_ad, w_h1, b_h1,
      w_h2, b_h2)


def _tc_head2(logits, m, s, amin, sel):
    def body(lg_r, m_r, s_r, amin_r, sel_r, o_hard, o_soft):
        i = pl.program_id(0)
        lg = lg_r[...]
        rid = lax.broadcasted_iota(jnp.int32, (B, 32), 0) + i * B
        lm = jnp.where(rid < N_DEV, lg, -1e30)
        soft = jnp.exp(lm - m_r[...]) / s_r[...]
        onehot = (rid == amin_r[...]).astype(jnp.float32)
        o_soft[...] = _dot(soft, sel_r[...])
        o_hard[...] = (_dot(onehot, sel_r[...]) > 0).astype(jnp.float32)

    return pl.pallas_call(
        body,
        grid=(NP // B,),
        in_specs=[_rowblk(32), _full((1, 32)), _full((1, 32)),
                  _full((1, 32)), _full((32, 4))],
        out_specs=[_rowblk(4), _rowblk(4)],
        out_shape=[jax.ShapeDtypeStruct((NP, 4), jnp.float32),
                   jax.ShapeDtypeStruct((NP, 4), jnp.float32)],
    )(logits, m, s, amin, sel)


def kernel(device_pos, ap_pos, csi, params, node_packets, dd_edge_index,
           da_src, da_dst, ad_src, ad_dst):
    p = params
    f32 = jnp.float32

    x0 = jnp.transpose(csi, (1, 0, 4, 2, 3)).reshape(N_AP, N_DEV, 64)
    x0 = jnp.pad(x0, ((0, 0), (0, NP - N_DEV), (0, 0)))

    pos_in = jnp.concatenate(
        [device_pos, node_packets[:, None].astype(f32),
         jnp.zeros((N_DEV, 5), f32)], axis=1)
    pos_in = jnp.pad(pos_in, ((0, NP - N_DEV), (0, 0)))

    ap_pos_p = jnp.pad(ap_pos, ((0, APP - N_AP), (0, 6)))

    w1b = _conv_as_matrix(p['csi_conv1_w'], _C1_IDX, 2, 16)
    b1b = jnp.repeat(p['csi_conv1_b'], 32)[None, :]
    w2b = _conv_as_matrix(p['csi_conv2_w'], _C2_IDX, 16, 32)
    b2b = jnp.repeat(p['csi_conv2_b'], 32)[None, :]
    wdp = jnp.pad(p['dev_pos_w'], ((0, 5), (0, 0)))
    wap = jnp.pad(p['ap_enc_w'], ((0, 6), (0, 0)))

    dd_s = _pad_edges(dd_edge_index[0], DD_P, PAD_SRC_DEV)
    dd_d = _pad_edges(dd_edge_index[1], DD_P, PAD_DST_DEV)
    da_s = _pad_edges(da_src, DA_P, PAD_SRC_DEV)
    da_d = _pad_edges(da_dst, DA_P, PAD_AP)
    ad_s = _pad_edges(ad_src, DA_P, PAD_AP)
    ad_d = _pad_edges(ad_dst, DA_P, PAD_DST_DEV)

    zeros8 = jnp.zeros((NP, 8), f32)
    zeros64 = jnp.zeros((NP, 64), f32)
    ones8 = jnp.ones((CHUNK, 8), f32)
    sel = jnp.asarray(_S_NP)

    h_dds, h_ddd, h_das, h_add, h_dad, h_ads = _sc_histograms(
        dd_s, dd_d, da_s, da_d, ad_s, ad_d, zeros8, ones8)

    h_dev = _tc_encoder(
        x0, pos_in, w1b, b1b, w2b, b2b,
        p['csi_fc_w'], p['csi_fc_b'][None, :],
        p['dev_csi_w'], p['dev_csi_b'][None, :],
        wdp, p['dev_pos_b'][None, :],
        p['dev_merge_w'][:16], p['dev_merge_w'][16:],
        p['dev_merge_b'][None, :])

    f_dd1, f_da1, f_ad1 = _tc_feat1(
        h_dev, ap_pos_p, h_dds, h_das, h_ads, wap, p['ap_enc_b'][None, :])
    agg_dd1, agg_ad1, agg_da1 = _sc_segsum(
        f_dd1, f_da1, f_ad1, dd_s, dd_d, da_s, da_d, ad_s, ad_d, zeros64,
        include_da=True)

    f_dd2, f_da2, f_ad2 = _tc_comb1(
        agg_dd1, agg_ad1, agg_da1, h_ddd, h_add, h_dad, h_dds, h_das, h_ads,
        p['c1_dd_w'], p['c1_ad_w'], p['c1_da_w'],
        p['c1_dd_b'][None, :], p['c1_ad_b'][None, :], p['c1_da_b'][None, :])
    agg_dd2, agg_ad2 = _sc_segsum(
        f_dd2, f_da2, f_ad2, dd_s, dd_d, da_s, da_d, ad_s, ad_d, zeros64,
        include_da=False)

    logits_p, m, s, amin = _tc_head1(
        agg_dd2, agg_ad2, h_ddd, h_add,
        p['c2_dd_w'], p['c2_ad_w'],
        p['c2_dd_b'][None, :], p['c2_ad_b'][None, :],
        p['aph1_w'], p['aph1_b'][None, :], p['aph2_w'], p['aph2_b'][None, :])
    hard_p, soft_p = _tc_head2(logits_p, m, s, amin, sel)

    sched_hard = hard_p[:N_DEV]
    ap_logits = logits_p[:N_DEV].reshape(N_DEV, 4, 8)
    sched_soft = soft_p[:N_DEV]
    return sched_hard, ap_logits, sched_soft
